# Initial kernel scaffold; baseline (speedup 1.0000x reference)
import jax
import jax.numpy as jnp
from jax.experimental import pallas as pl


def kernel(x, edge_index, W1, b1, W2, b2, We, be, Wl, bl):
    raise NotImplementedError("write your pallas kernel here")


# speedup vs baseline: 8.0403x; 8.0403x over previous
import functools

import jax
import jax.numpy as jnp
from jax import lax
from jax.experimental import pallas as pl
from jax.experimental.pallas import tpu as pltpu
from jax.experimental.pallas import tpu_sc as plsc

N = 10000
E = 320000
D = 128
H = 128
L = 16

NSUB = 16
NTILES = 2 * NSUB
NPAD = 10240
RPT = NPAD // NSUB
CH = 128
NCHUNK = (E // NTILES + CH - 1) // CH
EPT = NCHUNK * CH
EPAD = EPT * NTILES

_mesh = plsc.VectorSubcoreMesh(core_axis_name="c", subcore_axis_name="s")


DEGW = 128


@functools.partial(
    pl.kernel,
    mesh=_mesh,
    out_type=jax.ShapeDtypeStruct((2, NPAD, DEGW), jnp.float32),
    scratch_types=[
        pltpu.VMEM((CH,), jnp.int32),
        pltpu.VMEM((CH, DEGW), jnp.float32),
        pltpu.VMEM_SHARED((NPAD, DEGW), jnp.float32),
    ],
)
def _deg_kernel(dst_hbm, ones_hbm, zeros_hbm, out_hbm, idx_v, ones_v, acc_sh):
    cid = lax.axis_index("c")
    sid = lax.axis_index("s")
    wid = cid * NSUB + sid
    pltpu.sync_copy(zeros_hbm.at[pl.ds(sid * RPT, RPT)],
                    acc_sh.at[pl.ds(sid * RPT, RPT)])
    pltpu.sync_copy(ones_hbm, ones_v)
    plsc.subcore_barrier()

    def body(i, carry):
        base = wid * EPT + i * CH
        pltpu.sync_copy(dst_hbm.at[pl.ds(base, CH)], idx_v)
        pltpu.sync_copy(ones_v, acc_sh.at[idx_v], add=True)
        return carry

    lax.fori_loop(0, NCHUNK, body, 0)
    plsc.subcore_barrier()
    pltpu.sync_copy(acc_sh.at[pl.ds(sid * RPT, RPT)],
                    out_hbm.at[cid, pl.ds(sid * RPT, RPT)])


@functools.partial(
    pl.kernel,
    mesh=_mesh,
    out_type=jax.ShapeDtypeStruct((2, NPAD, H), jnp.float32),
    scratch_types=[
        pltpu.VMEM((CH,), jnp.int32),
        pltpu.VMEM((CH,), jnp.int32),
        pltpu.VMEM((CH, H), jnp.float32),
        pltpu.VMEM_SHARED((NPAD, H), jnp.float32),
        pltpu.SemaphoreType.DMA,
    ],
)
def _mp_kernel(u_hbm, src_hbm, dst_hbm, zeros_hbm, out_hbm,
               sidx_v, didx_v, rows_v, acc_sh, sem):
    cid = lax.axis_index("c")
    sid = lax.axis_index("s")
    wid = cid * NSUB + sid
    pltpu.sync_copy(zeros_hbm.at[pl.ds(sid * RPT, RPT)],
                    acc_sh.at[pl.ds(sid * RPT, RPT)])
    plsc.subcore_barrier()

    def body(i, carry):
        base = wid * EPT + i * CH
        pltpu.sync_copy(src_hbm.at[pl.ds(base, CH)], sidx_v)
        pltpu.sync_copy(dst_hbm.at[pl.ds(base, CH)], didx_v)
        pltpu.async_copy(u_hbm.at[sidx_v], rows_v, sem).wait()
        pltpu.sync_copy(rows_v, acc_sh.at[didx_v], add=True)
        return carry

    lax.fori_loop(0, NCHUNK, body, 0)
    plsc.subcore_barrier()
    pltpu.sync_copy(acc_sh.at[pl.ds(sid * RPT, RPT)],
                    out_hbm.at[cid, pl.ds(sid * RPT, RPT)])


RB = 400
GRID = N // RB


def _prep1_body(deg_ref, x_ref, w_ref, dinv_ref, u_ref):
    deg = deg_ref[0, :, 0:1] + deg_ref[1, :, 0:1] + 2.0
    dinv = lax.rsqrt(deg)
    h = jnp.dot(x_ref[...], w_ref[...], preferred_element_type=jnp.float32,
                precision=lax.Precision.HIGHEST)
    dinv_ref[...] = dinv
    u_ref[...] = h * dinv


def _prep1(deg, x, w1):
    return pl.pallas_call(
        _prep1_body,
        grid=(GRID,),
        in_specs=[
            pl.BlockSpec((2, RB, DEGW), lambda i: (0, i, 0)),
            pl.BlockSpec((RB, D), lambda i: (i, 0)),
            pl.BlockSpec((D, H), lambda i: (0, 0)),
        ],
        out_specs=[
            pl.BlockSpec((RB, 1), lambda i: (i, 0)),
            pl.BlockSpec((RB, H), lambda i: (i, 0)),
        ],
        out_shape=[
            jax.ShapeDtypeStruct((N, 1), jnp.float32),
            jax.ShapeDtypeStruct((N, H), jnp.float32),
        ],
    )(deg, x, w1)


def _fin_prep_body(s_ref, u_ref, dinv_ref, b_ref, w_ref, unext_ref):
    dinv = dinv_ref[...]
    s = s_ref[0] + s_ref[1]
    h = jnp.maximum(s * dinv + 2.0 * dinv * u_ref[...] + b_ref[...], 0.0)
    unext_ref[...] = jnp.dot(
        h, w_ref[...], preferred_element_type=jnp.float32,
        precision=lax.Precision.HIGHEST) * dinv


def _fin_prep(s, u, dinv, b, w):
    return pl.pallas_call(
        _fin_prep_body,
        grid=(GRID,),
        in_specs=[
            pl.BlockSpec((2, RB, H), lambda i: (0, i, 0)),
            pl.BlockSpec((RB, H), lambda i: (i, 0)),
            pl.BlockSpec((RB, 1), lambda i: (i, 0)),
            pl.BlockSpec((1, H), lambda i: (0, 0)),
            pl.BlockSpec((H, H), lambda i: (0, 0)),
        ],
        out_specs=pl.BlockSpec((RB, H), lambda i: (i, 0)),
        out_shape=jax.ShapeDtypeStruct((N, H), jnp.float32),
    )(s, u, dinv, b, w)


def _final_body(s_ref, u_ref, dinv_ref, be_ref, wl_ref, bl_ref, out_ref):
    dinv = dinv_ref[...]
    s = s_ref[0] + s_ref[1]
    h = s * dinv + 2.0 * dinv * u_ref[...] + be_ref[...]
    z = jnp.dot(h, wl_ref[...], preferred_element_type=jnp.float32,
                precision=lax.Precision.HIGHEST) + bl_ref[...]
    out_ref[...] = 1.0 / (1.0 + jnp.exp(-z))


def _final(s, u, dinv, be, wl, bl):
    return pl.pallas_call(
        _final_body,
        grid=(GRID,),
        in_specs=[
            pl.BlockSpec((2, RB, H), lambda i: (0, i, 0)),
            pl.BlockSpec((RB, H), lambda i: (i, 0)),
            pl.BlockSpec((RB, 1), lambda i: (i, 0)),
            pl.BlockSpec((1, H), lambda i: (0, 0)),
            pl.BlockSpec((H, L), lambda i: (0, 0)),
            pl.BlockSpec((1, L), lambda i: (0, 0)),
        ],
        out_specs=pl.BlockSpec((RB, L), lambda i: (i, 0)),
        out_shape=jax.ShapeDtypeStruct((N, L), jnp.float32),
    )(s, u, dinv, be, wl, bl)


def kernel(x, edge_index, W1, b1, W2, b2, We, be, Wl, bl):
    ei = edge_index.astype(jnp.int32)
    pad = EPAD - E
    src_p = jnp.concatenate([ei[0], jnp.zeros((pad,), jnp.int32)])
    dst_p = jnp.concatenate([ei[1], jnp.full((pad,), N, jnp.int32)])

    ones1 = jnp.ones((CH, DEGW), jnp.float32)
    z1 = jnp.zeros((NPAD, DEGW), jnp.float32)
    z2 = jnp.zeros((NPAD, H), jnp.float32)

    deg = _deg_kernel(dst_p, ones1, z1)
    dinv, u1 = _prep1(deg, x, W1)
    s1 = _mp_kernel(u1, src_p, dst_p, z2)
    u2 = _fin_prep(s1, u1, dinv, b1.reshape(1, H), W2)
    s2 = _mp_kernel(u2, src_p, dst_p, z2)
    u3 = _fin_prep(s2, u2, dinv, b2.reshape(1, H), We)
    s3 = _mp_kernel(u3, src_p, dst_p, z2)
    return _final(s3, u3, dinv, be.reshape(1, H), Wl, bl.reshape(1, L))
